# baseline (speedup 1.0000x reference)
import functools

import jax
import jax.numpy as jnp
from jax.experimental import pallas as pl
from jax.experimental.pallas import tpu as pltpu


def _scan_tile(mb, qn, acc):
    mbb = mb.astype(jnp.float8_e4m3fn)
    m2 = jnp.sum(mb * mb, axis=1, keepdims=True)
    inv = jax.lax.rsqrt(m2)
    dot = jax.lax.dot_general(
        mbb, qn,
        (((1,), (1,)), ((), ())), preferred_element_type=jnp.float32)
    return jnp.maximum(acc, jnp.max(inv * dot, axis=0, keepdims=True))


def _body(q_ref, mba_ref, mbb_ref, out_ref, qn_ref, q2_ref, acc_ref,
          *, ntiles, B, P):
    step = pl.program_id(0)
    Q = B * P

    @pl.when(step == 0)
    def _init():
        q = q_ref[...]
        qnorm = jnp.sqrt(jnp.sum(q * q, axis=1, keepdims=True))
        qn = q / (qnorm + 1e-12)
        qn_ref[...] = qn.astype(jnp.float8_e4m3fn)
        q2_ref[...] = jax.lax.dot_general(
            jnp.ones((1, q.shape[1]), jnp.float32), qn * qn,
            (((1,), (1,)), ((), ())), preferred_element_type=jnp.float32)
        acc_ref[...] = jnp.full((1, Q), -jnp.inf, jnp.float32)

    acc = _scan_tile(mba_ref[...], qn_ref[...], acc_ref[...])
    acc_ref[...] = _scan_tile(mbb_ref[...], qn_ref[...], acc)

    @pl.when(step == ntiles - 1)
    def _fin():
        d2 = q2_ref[...] + 1.0 - 2.0 * acc_ref[...]
        d = jnp.sqrt(jnp.maximum(d2, 0.0) + 1e-12)
        col = jax.lax.broadcasted_iota(jnp.int32, (B, Q), 1)
        row = jax.lax.broadcasted_iota(jnp.int32, (B, Q), 0)
        mask = (col >= row * P) & (col < (row + 1) * P)
        db = jnp.broadcast_to(d, (B, Q))
        out_ref[...] = jnp.max(jnp.where(mask, db, -jnp.inf), axis=1,
                               keepdims=True)


def kernel(queries, memory_bank):
    B, P, D = queries.shape
    M = memory_bank.shape[0]
    Q = B * P
    HALF = 2000
    assert M % (2 * HALF) == 0
    ntiles = M // (2 * HALF)

    qf = queries.reshape(Q, D)
    out = pl.pallas_call(
        functools.partial(_body, ntiles=ntiles, B=B, P=P),
        grid=(ntiles,),
        in_specs=[
            pl.BlockSpec((Q, D), lambda i: (0, 0)),
            pl.BlockSpec((HALF, D), lambda i: (2 * i, 0)),
            pl.BlockSpec((HALF, D), lambda i: (2 * i + 1, 0)),
        ],
        out_specs=pl.BlockSpec((B, 1), lambda i: (0, 0)),
        out_shape=jax.ShapeDtypeStruct((B, 1), jnp.float32),
        scratch_shapes=[
            pltpu.VMEM((Q, D), jnp.float8_e4m3fn),
            pltpu.VMEM((1, Q), jnp.float32),
            pltpu.VMEM((1, Q), jnp.float32),
        ],
        compiler_params=pltpu.CompilerParams(
            dimension_semantics=("arbitrary",)),
    )(qf, memory_bank, memory_bank)
    return out.reshape(B)


# speedup vs baseline: 1.0335x; 1.0335x over previous
import functools

import jax
import jax.numpy as jnp
from jax.experimental import pallas as pl
from jax.experimental.pallas import tpu as pltpu


def _body(q_ref, mb_ref, out_ref, qn_ref, q2_ref, acc_ref, *, ntiles, B, P):
    step = pl.program_id(0)
    Q = B * P

    @pl.when(step == 0)
    def _init():
        q = q_ref[...]
        qnorm = jnp.sqrt(jnp.sum(q * q, axis=1, keepdims=True))
        qn = q / (qnorm + 1e-12)
        qn_ref[...] = qn.astype(jnp.float8_e4m3fn)
        q2_ref[...] = jax.lax.dot_general(
            jnp.ones((1, q.shape[1]), jnp.float32), qn * qn,
            (((1,), (1,)), ((), ())), preferred_element_type=jnp.float32)
        acc_ref[...] = jnp.full((1, Q), -jnp.inf, jnp.float32)

    mb = mb_ref[...]
    mbb = mb.astype(jnp.float8_e4m3fn)
    m2 = jnp.sum(mb * mb, axis=1, keepdims=True)
    inv = jax.lax.rsqrt(m2)
    dot = jax.lax.dot_general(
        mbb, qn_ref[...],
        (((1,), (1,)), ((), ())), preferred_element_type=jnp.float32)
    acc_ref[...] = jnp.maximum(acc_ref[...],
                               jnp.max(inv * dot, axis=0, keepdims=True))

    @pl.when(step == ntiles - 1)
    def _fin():
        d2 = q2_ref[...] + 1.0 - 2.0 * acc_ref[...]
        d = jnp.sqrt(jnp.maximum(d2, 0.0) + 1e-12)
        col = jax.lax.broadcasted_iota(jnp.int32, (B, Q), 1)
        row = jax.lax.broadcasted_iota(jnp.int32, (B, Q), 0)
        mask = (col >= row * P) & (col < (row + 1) * P)
        db = jnp.broadcast_to(d, (B, Q))
        out_ref[...] = jnp.max(jnp.where(mask, db, -jnp.inf), axis=1,
                               keepdims=True)


def kernel(queries, memory_bank):
    B, P, D = queries.shape
    M = memory_bank.shape[0]
    Q = B * P
    TILE = 5000
    assert M % TILE == 0
    ntiles = M // TILE

    qf = queries.reshape(Q, D)
    out = pl.pallas_call(
        functools.partial(_body, ntiles=ntiles, B=B, P=P),
        grid=(ntiles,),
        in_specs=[
            pl.BlockSpec((Q, D), lambda i: (0, 0)),
            pl.BlockSpec((TILE, D), lambda i: (i, 0)),
        ],
        out_specs=pl.BlockSpec((B, 1), lambda i: (0, 0)),
        out_shape=jax.ShapeDtypeStruct((B, 1), jnp.float32),
        scratch_shapes=[
            pltpu.VMEM((Q, D), jnp.float8_e4m3fn),
            pltpu.VMEM((1, Q), jnp.float32),
            pltpu.VMEM((1, Q), jnp.float32),
        ],
        compiler_params=pltpu.CompilerParams(
            dimension_semantics=("arbitrary",)),
    )(qf, memory_bank)
    return out.reshape(B)
